# Initial kernel scaffold; baseline (speedup 1.0000x reference)
import jax
import jax.numpy as jnp
from jax.experimental import pallas as pl


def kernel(x_var, edge_src, edge_dst, W_enc, b_enc, W_time, W_dec, b_dec, W_lin, b_lin, W_att1, b_att1, W_att2, b_att2, W_out, b_out):
    raise NotImplementedError("write your pallas kernel here")


# speedup vs baseline: 10.1668x; 10.1668x over previous
import functools

import jax
import jax.numpy as jnp
from jax import lax
from jax.experimental import pallas as pl
from jax.experimental.pallas import tpu as pltpu
from jax.experimental.pallas import tpu_sc as plsc

NV = 50000
NHP = 12288
NE = 800000
HD = 64
TT = 4

NC = 2
NS = 16

NV_PAD = 51200
NHP_PAD = 12544
E_PAD = 802816
ER = E_PAD // 128


def _sc_mesh():
    return plsc.VectorSubcoreMesh(core_axis_name="c", subcore_axis_name="s")


@functools.cache
def _get_deg_kernel():
    return functools.partial(
        pl.kernel,
        out_type=jax.ShapeDtypeStruct((NC, NV_PAD, 16), jnp.float32),
        mesh=_sc_mesh(),
        compiler_params=pltpu.CompilerParams(use_tc_tiling_on_sc=False),
        scratch_types=[
            pltpu.VMEM((4, 128), jnp.int32),
            pltpu.VMEM((128, 16), jnp.float32),
            pltpu.VMEM((3200, 16), jnp.float32),
            pltpu.VMEM_SHARED((NV_PAD, 16), jnp.float32),
        ],
    )(_deg_body)


def _deg_body(src_hbm, ones_hbm, z3_hbm, ds_out, sidx, ones, zbuf, accs):
    c = lax.axis_index("c")
    s = lax.axis_index("s")
    wid = s * NC + c
    pltpu.sync_copy(ones_hbm, ones)
    pltpu.sync_copy(z3_hbm, zbuf)
    pltpu.sync_copy(zbuf, accs.at[pl.ds(s * 3200, 3200)])
    plsc.subcore_barrier()
    base = wid * 196

    def body(i, _):
        r0 = base + i * 4
        pltpu.sync_copy(src_hbm.at[pl.ds(r0, 4)], sidx)
        for j in range(4):
            pltpu.sync_copy(ones, accs.at[sidx.at[j]], add=True)
        return 0

    lax.fori_loop(0, 49, body, 0)
    plsc.subcore_barrier()
    pltpu.sync_copy(accs.at[pl.ds(s * 3200, 3200)],
                    ds_out.at[c, pl.ds(s * 3200, 3200)])


@functools.cache
def _get_enc_kernel():
    return functools.partial(
        pl.kernel,
        out_type=jax.ShapeDtypeStruct((NC, NHP_PAD, 16), jnp.float32),
        mesh=_sc_mesh(),
        compiler_params=pltpu.CompilerParams(use_tc_tiling_on_sc=False),
        scratch_types=[
            pltpu.VMEM((4, 128), jnp.int32),
            pltpu.VMEM((4, 128), jnp.int32),
            pltpu.VMEM((128, 16), jnp.float32),
            pltpu.VMEM((784, 16), jnp.float32),
            pltpu.VMEM_SHARED((NHP_PAD, 16), jnp.float32),
            pltpu.SemaphoreType.DMA,
        ],
    )(_enc_body)


def _enc_body(xs_hbm, src_hbm, dst_hbm, z2_hbm, enc_out, sidx, didx, rbuf,
              zbuf, acce, sem):
    c = lax.axis_index("c")
    s = lax.axis_index("s")
    wid = s * NC + c
    pltpu.sync_copy(z2_hbm, zbuf)
    pltpu.sync_copy(zbuf, acce.at[pl.ds(s * 784, 784)])
    plsc.subcore_barrier()
    base = wid * 196

    def body(i, _):
        r0 = base + i * 4
        pltpu.sync_copy(src_hbm.at[pl.ds(r0, 4)], sidx)
        pltpu.sync_copy(dst_hbm.at[pl.ds(r0, 4)], didx)
        for j in range(4):
            pltpu.async_copy(xs_hbm.at[sidx.at[j]], rbuf, sem).wait()
            pltpu.sync_copy(rbuf, acce.at[didx.at[j]], add=True)
        return 0

    lax.fori_loop(0, 49, body, 0)
    plsc.subcore_barrier()
    pltpu.sync_copy(acce.at[pl.ds(s * 784, 784)],
                    enc_out.at[c, pl.ds(s * 784, 784)])


@functools.cache
def _get_dec_kernel():
    return functools.partial(
        pl.kernel,
        out_type=jax.ShapeDtypeStruct((16, NV_PAD, 16), jnp.float32),
        mesh=_sc_mesh(),
        compiler_params=pltpu.CompilerParams(use_tc_tiling_on_sc=False),
        scratch_types=[
            pltpu.VMEM((4, 128), jnp.int32),
            pltpu.VMEM((4, 128), jnp.int32),
            pltpu.VMEM((128, 16), jnp.float32),
            pltpu.VMEM((3200, 16), jnp.float32),
            pltpu.VMEM_SHARED((NV_PAD, 16), jnp.float32),
            pltpu.SemaphoreType.DMA,
        ],
    )(_dec_body)


def _dec_body(y_hbm, src_hbm, dst_hbm, z3_hbm, dec_out, sidx, didx, rbuf,
              zbuf, acc, sem):
    c = lax.axis_index("c")
    s = lax.axis_index("s")
    pltpu.sync_copy(z3_hbm, zbuf)
    for cc_local in range(8):
        cc = c * 8 + cc_local
        pltpu.sync_copy(zbuf, acc.at[pl.ds(s * 3200, 3200)])
        plsc.subcore_barrier()
        base = s * 392

        def body(i, _):
            r0 = base + i * 4
            pltpu.sync_copy(src_hbm.at[pl.ds(r0, 4)], sidx)
            pltpu.sync_copy(dst_hbm.at[pl.ds(r0, 4)], didx)
            for j in range(4):
                pltpu.async_copy(y_hbm.at[cc].at[didx.at[j]], rbuf, sem).wait()
                pltpu.sync_copy(rbuf, acc.at[sidx.at[j]], add=True)
            return 0

        lax.fori_loop(0, 98, body, 0)
        plsc.subcore_barrier()
        pltpu.sync_copy(acc.at[pl.ds(s * 3200, 3200)],
                        dec_out.at[cc, pl.ds(s * 3200, 3200)])
        plsc.subcore_barrier()


def _xs_body(degp_ref, xvt_ref, out_ref, iss_ref):
    deg = degp_ref[0, :, 0] + degp_ref[1, :, 0]
    is_src = lax.rsqrt(jnp.maximum(deg, 1.0))
    x = jnp.nan_to_num(xvt_ref[...])
    b = x.shape[0]
    xsc = x * is_src[:, None]
    out_ref[...] = jnp.concatenate(
        [xsc, jnp.ones((b, 1), jnp.float32),
         jnp.zeros((b, 11), jnp.float32)], axis=1)
    iss_ref[...] = is_src[:, None]


def _tc_xs(degp, xvt):
    b = 3200
    return pl.pallas_call(
        _xs_body,
        out_shape=(jax.ShapeDtypeStruct((NV_PAD, 16), jnp.float32),
                   jax.ShapeDtypeStruct((NV_PAD, 1), jnp.float32)),
        grid=(NV_PAD // b,),
        in_specs=[
            pl.BlockSpec((NC, b, 16), lambda i: (0, i, 0)),
            pl.BlockSpec((b, TT), lambda i: (i, 0)),
        ],
        out_specs=(pl.BlockSpec((b, 16), lambda i: (i, 0)),
                   pl.BlockSpec((b, 1), lambda i: (i, 0))),
    )(degp, xvt)


def _y_body(encp_ref, wenc_ref, benc_ref, wtime_ref, out_ref):
    ep = encp_ref[0] + encp_ref[1]
    agg = ep[:, :TT]
    degd = ep[:, TT]
    isd = lax.rsqrt(jnp.maximum(degd, 1.0))
    a = agg * isd[:, None]
    wenc = wenc_ref[...]
    benc = benc_ref[...]
    wt = wtime_ref[...]
    hs = []
    for t in range(TT):
        h = a[:, t][:, None] * wenc[t][None, :] + benc[t][None, :]
        hs.append(jnp.where(h > 0, h, 0.01 * h))
    for o in range(TT):
        y = hs[0] * wt[0, o]
        for t in range(1, TT):
            y = y + hs[t] * wt[t, o]
        y = y * isd[:, None]
        for k in range(4):
            out_ref[4 * o + k] = y[:, 16 * k:16 * (k + 1)]


def _tc_y(encp, wenc, benc, wtime):
    b = 1792
    return pl.pallas_call(
        _y_body,
        out_shape=jax.ShapeDtypeStruct((16, NHP_PAD, 16), jnp.float32),
        grid=(NHP_PAD // b,),
        in_specs=[
            pl.BlockSpec((NC, b, 16), lambda i: (0, i, 0)),
            pl.BlockSpec((TT, HD), lambda i: (0, 0)),
            pl.BlockSpec((TT, HD), lambda i: (0, 0)),
            pl.BlockSpec((TT, TT), lambda i: (0, 0)),
        ],
        out_specs=pl.BlockSpec((16, b, 16), lambda i: (0, i, 0)),
    )(encp, wenc, benc, wtime)


def _out_body(dec_ref, iss_ref, xvt_ref, wdec_ref, bdec_ref, wlin_ref,
              blin_ref, watt1_ref, batt1_ref, watt2_ref, batt2_ref, wout_ref,
              bout_ref, out_ref):
    iss = iss_ref[:, 0]
    x = jnp.nan_to_num(xvt_ref[...])
    watt1 = watt1_ref[...]
    batt1 = batt1_ref[...]
    watt2 = watt2_ref[...]
    batt2 = batt2_ref[0]
    wlin = wlin_ref[...]
    blin = blin_ref[...]
    wout = wout_ref[...]
    bout = bout_ref[0]
    outs = []
    for t in range(TT):
        agg = jnp.concatenate([dec_ref[4 * t + k] for k in range(4)],
                              axis=1)
        agg = agg * iss[:, None]
        s1 = jnp.dot(agg, wdec_ref[t], preferred_element_type=jnp.float32)
        s1 = s1 + bdec_ref[t][None, :]
        s1 = jnp.where(s1 > 0, s1, 0.01 * s1)
        s2 = x[:, t][:, None] * wlin[None, :] + blin[None, :]
        t1 = jnp.tanh(jnp.dot(s1, watt1, preferred_element_type=jnp.float32)
                      + batt1[None, :])
        t2 = jnp.tanh(jnp.dot(s2, watt1, preferred_element_type=jnp.float32)
                      + batt1[None, :])
        a1 = jnp.sum(t1 * watt2[None, :], axis=1) + batt2
        a2 = jnp.sum(t2 * watt2[None, :], axis=1) + batt2
        m = jnp.maximum(a1, a2)
        e1 = jnp.exp(a1 - m)
        e2 = jnp.exp(a2 - m)
        inv = 1.0 / (e1 + e2)
        al1 = (e1 * inv)[:, None]
        al2 = (e2 * inv)[:, None]
        pooled = al1 * s1 + al2 * s2
        outs.append(jnp.sum(pooled * wout[None, :], axis=1) + bout)
    out_ref[...] = jnp.stack(outs, axis=0)[:, :, None]


def _tc_out(dec, iss, xvt, wdec, bdec, wlin, blin, watt1, batt1, watt2,
            batt2, wout, bout):
    b = 1600
    full = lambda shape: pl.BlockSpec(shape, lambda i: tuple(0 for _ in shape))
    return pl.pallas_call(
        _out_body,
        out_shape=jax.ShapeDtypeStruct((TT, NV_PAD, 1), jnp.float32),
        grid=(NV_PAD // b,),
        in_specs=[
            pl.BlockSpec((16, b, 16), lambda i: (0, i, 0)),
            pl.BlockSpec((b, 1), lambda i: (i, 0)),
            pl.BlockSpec((b, TT), lambda i: (i, 0)),
            full((TT, HD, HD)),
            full((TT, HD)),
            full((HD,)),
            full((HD,)),
            full((HD, 2 * HD)),
            full((2 * HD,)),
            full((2 * HD,)),
            full((1,)),
            full((HD,)),
            full((1,)),
        ],
        out_specs=pl.BlockSpec((TT, b, 1), lambda i: (0, i, 0)),
    )(dec, iss, xvt, wdec, bdec, wlin, blin, watt1, batt1, watt2, batt2,
      wout, bout)


def kernel(x_var, edge_src, edge_dst, W_enc, b_enc, W_time, W_dec, b_dec,
           W_lin, b_lin, W_att1, b_att1, W_att2, b_att2, W_out, b_out):
    pad_n = E_PAD - NE
    pad_ids = jnp.arange(pad_n, dtype=jnp.int32)
    src2d = jnp.concatenate(
        [edge_src, NV + (pad_ids % 64)]).reshape(ER, 128)
    dst2d = jnp.concatenate(
        [edge_dst, NHP + (pad_ids % 64)]).reshape(ER, 128)
    xvt = jnp.zeros((NV_PAD, TT), jnp.float32)
    xvt = xvt.at[:NV].set(x_var[:, :, 0].T)
    z2 = jnp.zeros((784, 16), jnp.float32)
    z3 = jnp.zeros((3200, 16), jnp.float32)
    ones128 = jnp.ones((128, 16), jnp.float32)

    ds_p = _get_deg_kernel()(src2d, ones128, z3)
    xs, iss = _tc_xs(ds_p, xvt)
    enc_p = _get_enc_kernel()(xs, src2d, dst2d, z2)
    y = _tc_y(enc_p, W_enc[:, 0, :], b_enc, W_time)
    dec = _get_dec_kernel()(y, src2d, dst2d, z3)
    out = _tc_out(dec, iss, xvt, W_dec, b_dec, W_lin[0], b_lin, W_att1,
                  b_att1, W_att2[:, 0], b_att2, W_out[:, 0], b_out)
    return out[:, :NV]


# speedup vs baseline: 23.7720x; 2.3382x over previous
import functools

import jax
import jax.numpy as jnp
from jax import lax
from jax.experimental import pallas as pl
from jax.experimental.pallas import tpu as pltpu
from jax.experimental.pallas import tpu_sc as plsc

NV = 50000
NHP = 12288
NE = 800000
HD = 64
TT = 4

NC = 2
NS = 16

NV_PAD = 51200
NHP_PAD = 12544
E_PAD = 802816
ER = E_PAD // 128


def _sc_mesh():
    return plsc.VectorSubcoreMesh(core_axis_name="c", subcore_axis_name="s")


@functools.cache
def _get_deg_kernel():
    return functools.partial(
        pl.kernel,
        out_type=jax.ShapeDtypeStruct((NC, NV_PAD, 16), jnp.float32),
        mesh=_sc_mesh(),
        compiler_params=pltpu.CompilerParams(use_tc_tiling_on_sc=False),
        scratch_types=[
            pltpu.VMEM((2, 4, 128), jnp.int32),
            pltpu.VMEM((128, 16), jnp.float32),
            pltpu.VMEM((3200, 16), jnp.float32),
            pltpu.VMEM_SHARED((NV_PAD, 16), jnp.float32),
            pltpu.SemaphoreType.DMA,
            pltpu.SemaphoreType.DMA,
        ],
    )(_deg_body)


def _deg_body(src_hbm, ones_hbm, z3_hbm, ds_out, sidx, ones, zbuf, accs,
              semi, sems):
    c = lax.axis_index("c")
    s = lax.axis_index("s")
    wid = s * NC + c
    pltpu.sync_copy(ones_hbm, ones)
    pltpu.sync_copy(z3_hbm, zbuf)
    pltpu.sync_copy(zbuf, accs.at[pl.ds(s * 3200, 3200)])
    plsc.subcore_barrier()
    base = wid * 196
    pltpu.async_copy(src_hbm.at[pl.ds(base, 4)], sidx.at[0], semi)

    def body(g, _):
        h = g % 2
        pltpu.make_async_copy(src_hbm.at[pl.ds(0, 4)], sidx.at[0], semi).wait()
        for j in range(4):
            pltpu.async_copy(ones, accs.at[sidx.at[h, j]], sems, add=True)

        @pl.when(g >= 1)
        def _():
            for j in range(4):
                pltpu.make_async_copy(ones, accs.at[sidx.at[0, j]],
                                      sems).wait()

        @pl.when(g < 48)
        def _():
            pltpu.async_copy(src_hbm.at[pl.ds(base + (g + 1) * 4, 4)],
                             sidx.at[1 - h], semi)

        return 0

    lax.fori_loop(0, 49, body, 0)
    for j in range(4):
        pltpu.make_async_copy(ones, accs.at[sidx.at[0, j]], sems).wait()
    plsc.subcore_barrier()
    pltpu.sync_copy(accs.at[pl.ds(s * 3200, 3200)],
                    ds_out.at[c, pl.ds(s * 3200, 3200)])


@functools.cache
def _get_enc_kernel():
    return functools.partial(
        pl.kernel,
        out_type=jax.ShapeDtypeStruct((NC, NHP_PAD, 16), jnp.float32),
        mesh=_sc_mesh(),
        compiler_params=pltpu.CompilerParams(use_tc_tiling_on_sc=False),
        scratch_types=[
            pltpu.VMEM((4, 4, 128), jnp.int32),
            pltpu.VMEM((4, 4, 128), jnp.int32),
            pltpu.VMEM((2, 4, 128, 16), jnp.float32),
            pltpu.VMEM((784, 16), jnp.float32),
            pltpu.VMEM_SHARED((NHP_PAD, 16), jnp.float32),
            pltpu.SemaphoreType.DMA,
            pltpu.SemaphoreType.DMA,
            pltpu.SemaphoreType.DMA,
        ],
    )(_enc_body)


def _enc_body(xs_hbm, src_hbm, dst_hbm, z2_hbm, enc_out, sidx, didx, rbuf,
              zbuf, acce, semi, semg, sems):
    c = lax.axis_index("c")
    s = lax.axis_index("s")
    wid = s * NC + c
    pltpu.sync_copy(z2_hbm, zbuf)
    pltpu.sync_copy(zbuf, acce.at[pl.ds(s * 784, 784)])
    plsc.subcore_barrier()
    base = wid * 196
    pltpu.async_copy(src_hbm.at[pl.ds(base, 4)], sidx.at[0], semi)
    pltpu.async_copy(dst_hbm.at[pl.ds(base, 4)], didx.at[0], semi)

    def body(g, _):
        h4 = g % 4
        hr = g % 2
        pltpu.make_async_copy(src_hbm.at[pl.ds(0, 4)], sidx.at[0], semi).wait()
        pltpu.make_async_copy(dst_hbm.at[pl.ds(0, 4)], didx.at[0], semi).wait()

        @pl.when(g >= 2)
        def _():
            for j in range(4):
                pltpu.make_async_copy(rbuf.at[0, j],
                                      acce.at[sidx.at[0, j]], sems).wait()

        for j in range(4):
            pltpu.async_copy(xs_hbm.at[sidx.at[h4, j]], rbuf.at[hr, j], semg)

        @pl.when(g < 48)
        def _():
            r1 = base + (g + 1) * 4
            nb = (g + 1) % 4
            pltpu.async_copy(src_hbm.at[pl.ds(r1, 4)], sidx.at[nb], semi)
            pltpu.async_copy(dst_hbm.at[pl.ds(r1, 4)], didx.at[nb], semi)

        for j in range(4):
            pltpu.make_async_copy(xs_hbm.at[sidx.at[0, j]], rbuf.at[0, j],
                                  semg).wait()
        for j in range(4):
            pltpu.async_copy(rbuf.at[hr, j], acce.at[didx.at[h4, j]], sems,
                             add=True)
        return 0

    lax.fori_loop(0, 49, body, 0)
    for j in range(8):
        pltpu.make_async_copy(rbuf.at[0, j % 4],
                              acce.at[sidx.at[0, j % 4]], sems).wait()
    plsc.subcore_barrier()
    pltpu.sync_copy(acce.at[pl.ds(s * 784, 784)],
                    enc_out.at[c, pl.ds(s * 784, 784)])


@functools.cache
def _get_dec_kernel():
    return functools.partial(
        pl.kernel,
        out_type=jax.ShapeDtypeStruct((16, NV_PAD, 16), jnp.float32),
        mesh=_sc_mesh(),
        compiler_params=pltpu.CompilerParams(use_tc_tiling_on_sc=False),
        scratch_types=[
            pltpu.VMEM((4, 4, 128), jnp.int32),
            pltpu.VMEM((4, 4, 128), jnp.int32),
            pltpu.VMEM((2, 4, 128, 16), jnp.float32),
            pltpu.VMEM((3200, 16), jnp.float32),
            pltpu.VMEM_SHARED((NV_PAD, 16), jnp.float32),
            pltpu.SemaphoreType.DMA,
            pltpu.SemaphoreType.DMA,
            pltpu.SemaphoreType.DMA,
        ],
    )(_dec_body)


def _dec_body(y_hbm, src_hbm, dst_hbm, z3_hbm, dec_out, sidx, didx, rbuf,
              zbuf, acc, semi, semg, sems):
    c = lax.axis_index("c")
    s = lax.axis_index("s")
    pltpu.sync_copy(z3_hbm, zbuf)
    for cc_local in range(8):
        cc = c * 8 + cc_local
        pltpu.sync_copy(zbuf, acc.at[pl.ds(s * 3200, 3200)])
        plsc.subcore_barrier()
        base = s * 392
        pltpu.async_copy(src_hbm.at[pl.ds(base, 4)], sidx.at[0], semi)
        pltpu.async_copy(dst_hbm.at[pl.ds(base, 4)], didx.at[0], semi)

        def body(g, _):
            h4 = g % 4
            hr = g % 2
            pltpu.make_async_copy(src_hbm.at[pl.ds(0, 4)], sidx.at[0],
                                  semi).wait()
            pltpu.make_async_copy(dst_hbm.at[pl.ds(0, 4)], didx.at[0],
                                  semi).wait()

            @pl.when(g >= 2)
            def _():
                for j in range(4):
                    pltpu.make_async_copy(rbuf.at[0, j],
                                          acc.at[sidx.at[0, j]], sems).wait()

            for j in range(4):
                pltpu.async_copy(y_hbm.at[cc].at[didx.at[h4, j]],
                                 rbuf.at[hr, j], semg)

            @pl.when(g < 97)
            def _():
                r1 = base + (g + 1) * 4
                nb = (g + 1) % 4
                pltpu.async_copy(src_hbm.at[pl.ds(r1, 4)], sidx.at[nb],
                                 semi)
                pltpu.async_copy(dst_hbm.at[pl.ds(r1, 4)], didx.at[nb],
                                 semi)

            for j in range(4):
                pltpu.make_async_copy(y_hbm.at[cc].at[didx.at[0, j]],
                                      rbuf.at[0, j], semg).wait()
            for j in range(4):
                pltpu.async_copy(rbuf.at[hr, j], acc.at[sidx.at[h4, j]], sems,
                                 add=True)
            return 0

        lax.fori_loop(0, 98, body, 0)
        for j in range(8):
            pltpu.make_async_copy(rbuf.at[0, j % 4],
                                  acc.at[sidx.at[0, j % 4]], sems).wait()
        plsc.subcore_barrier()
        pltpu.sync_copy(acc.at[pl.ds(s * 3200, 3200)],
                        dec_out.at[cc, pl.ds(s * 3200, 3200)])
        plsc.subcore_barrier()


def _xs_body(degp_ref, xvt_ref, out_ref, iss_ref):
    deg = degp_ref[0, :, 0] + degp_ref[1, :, 0]
    is_src = lax.rsqrt(jnp.maximum(deg, 1.0))
    x = jnp.nan_to_num(xvt_ref[...])
    b = x.shape[0]
    xsc = x * is_src[:, None]
    out_ref[...] = jnp.concatenate(
        [xsc, jnp.ones((b, 1), jnp.float32),
         jnp.zeros((b, 11), jnp.float32)], axis=1)
    iss_ref[...] = is_src[:, None]


def _tc_xs(degp, xvt):
    b = 3200
    return pl.pallas_call(
        _xs_body,
        out_shape=(jax.ShapeDtypeStruct((NV_PAD, 16), jnp.float32),
                   jax.ShapeDtypeStruct((NV_PAD, 1), jnp.float32)),
        grid=(NV_PAD // b,),
        in_specs=[
            pl.BlockSpec((NC, b, 16), lambda i: (0, i, 0)),
            pl.BlockSpec((b, TT), lambda i: (i, 0)),
        ],
        out_specs=(pl.BlockSpec((b, 16), lambda i: (i, 0)),
                   pl.BlockSpec((b, 1), lambda i: (i, 0))),
    )(degp, xvt)


def _y_body(encp_ref, wenc_ref, benc_ref, wtime_ref, out_ref):
    ep = encp_ref[0] + encp_ref[1]
    agg = ep[:, :TT]
    degd = ep[:, TT]
    isd = lax.rsqrt(jnp.maximum(degd, 1.0))
    a = agg * isd[:, None]
    wenc = wenc_ref[...]
    benc = benc_ref[...]
    wt = wtime_ref[...]
    hs = []
    for t in range(TT):
        h = a[:, t][:, None] * wenc[t][None, :] + benc[t][None, :]
        hs.append(jnp.where(h > 0, h, 0.01 * h))
    for o in range(TT):
        y = hs[0] * wt[0, o]
        for t in range(1, TT):
            y = y + hs[t] * wt[t, o]
        y = y * isd[:, None]
        for k in range(4):
            out_ref[4 * o + k] = y[:, 16 * k:16 * (k + 1)]


def _tc_y(encp, wenc, benc, wtime):
    b = 1792
    return pl.pallas_call(
        _y_body,
        out_shape=jax.ShapeDtypeStruct((16, NHP_PAD, 16), jnp.float32),
        grid=(NHP_PAD // b,),
        in_specs=[
            pl.BlockSpec((NC, b, 16), lambda i: (0, i, 0)),
            pl.BlockSpec((TT, HD), lambda i: (0, 0)),
            pl.BlockSpec((TT, HD), lambda i: (0, 0)),
            pl.BlockSpec((TT, TT), lambda i: (0, 0)),
        ],
        out_specs=pl.BlockSpec((16, b, 16), lambda i: (0, i, 0)),
    )(encp, wenc, benc, wtime)


def _out_body(dec_ref, iss_ref, xvt_ref, wdec_ref, bdec_ref, wlin_ref,
              blin_ref, watt1_ref, batt1_ref, watt2_ref, batt2_ref, wout_ref,
              bout_ref, out_ref):
    iss = iss_ref[:, 0]
    x = jnp.nan_to_num(xvt_ref[...])
    watt1 = watt1_ref[...]
    batt1 = batt1_ref[...]
    watt2 = watt2_ref[...]
    batt2 = batt2_ref[0]
    wlin = wlin_ref[...]
    blin = blin_ref[...]
    wout = wout_ref[...]
    bout = bout_ref[0]
    outs = []
    for t in range(TT):
        agg = jnp.concatenate([dec_ref[4 * t + k] for k in range(4)],
                              axis=1)
        agg = agg * iss[:, None]
        s1 = jnp.dot(agg, wdec_ref[t], preferred_element_type=jnp.float32)
        s1 = s1 + bdec_ref[t][None, :]
        s1 = jnp.where(s1 > 0, s1, 0.01 * s1)
        s2 = x[:, t][:, None] * wlin[None, :] + blin[None, :]
        t1 = jnp.tanh(jnp.dot(s1, watt1, preferred_element_type=jnp.float32)
                      + batt1[None, :])
        t2 = jnp.tanh(jnp.dot(s2, watt1, preferred_element_type=jnp.float32)
                      + batt1[None, :])
        a1 = jnp.sum(t1 * watt2[None, :], axis=1) + batt2
        a2 = jnp.sum(t2 * watt2[None, :], axis=1) + batt2
        m = jnp.maximum(a1, a2)
        e1 = jnp.exp(a1 - m)
        e2 = jnp.exp(a2 - m)
        inv = 1.0 / (e1 + e2)
        al1 = (e1 * inv)[:, None]
        al2 = (e2 * inv)[:, None]
        pooled = al1 * s1 + al2 * s2
        outs.append(jnp.sum(pooled * wout[None, :], axis=1) + bout)
    out_ref[...] = jnp.stack(outs, axis=0)[:, :, None]


def _tc_out(dec, iss, xvt, wdec, bdec, wlin, blin, watt1, batt1, watt2,
            batt2, wout, bout):
    b = 1600
    full = lambda shape: pl.BlockSpec(shape, lambda i: tuple(0 for _ in shape))
    return pl.pallas_call(
        _out_body,
        out_shape=jax.ShapeDtypeStruct((TT, NV_PAD, 1), jnp.float32),
        grid=(NV_PAD // b,),
        in_specs=[
            pl.BlockSpec((16, b, 16), lambda i: (0, i, 0)),
            pl.BlockSpec((b, 1), lambda i: (i, 0)),
            pl.BlockSpec((b, TT), lambda i: (i, 0)),
            full((TT, HD, HD)),
            full((TT, HD)),
            full((HD,)),
            full((HD,)),
            full((HD, 2 * HD)),
            full((2 * HD,)),
            full((2 * HD,)),
            full((1,)),
            full((HD,)),
            full((1,)),
        ],
        out_specs=pl.BlockSpec((TT, b, 1), lambda i: (0, i, 0)),
    )(dec, iss, xvt, wdec, bdec, wlin, blin, watt1, batt1, watt2, batt2,
      wout, bout)


def kernel(x_var, edge_src, edge_dst, W_enc, b_enc, W_time, W_dec, b_dec,
           W_lin, b_lin, W_att1, b_att1, W_att2, b_att2, W_out, b_out):
    pad_n = E_PAD - NE
    pad_ids = jnp.arange(pad_n, dtype=jnp.int32)
    src2d = jnp.concatenate(
        [edge_src, NV + (pad_ids % 64)]).reshape(ER, 128)
    dst2d = jnp.concatenate(
        [edge_dst, NHP + (pad_ids % 64)]).reshape(ER, 128)
    xvt = jnp.zeros((NV_PAD, TT), jnp.float32)
    xvt = xvt.at[:NV].set(x_var[:, :, 0].T)
    z2 = jnp.zeros((784, 16), jnp.float32)
    z3 = jnp.zeros((3200, 16), jnp.float32)
    ones128 = jnp.ones((128, 16), jnp.float32)

    ds_p = _get_deg_kernel()(src2d, ones128, z3)
    xs, iss = _tc_xs(ds_p, xvt)
    enc_p = _get_enc_kernel()(xs, src2d, dst2d, z2)
    y = _tc_y(enc_p, W_enc[:, 0, :], b_enc, W_time)
    dec = _get_dec_kernel()(y, src2d, dst2d, z3)
    out = _tc_out(dec, iss, xvt, W_dec, b_dec, W_lin[0], b_lin, W_att1,
                  b_att1, W_att2[:, 0], b_att2, W_out[:, 0], b_out)
    return out[:, :NV]


# speedup vs baseline: 23.9452x; 1.0073x over previous
import functools

import jax
import jax.numpy as jnp
from jax import lax
from jax.experimental import pallas as pl
from jax.experimental.pallas import tpu as pltpu
from jax.experimental.pallas import tpu_sc as plsc

NV = 50000
NHP = 12288
NE = 800000
HD = 64
TT = 4

NC = 2
NS = 16

NV_PAD = 50176
NHP_PAD = 12544
E_PAD = 802816
ER = E_PAD // 128


def _sc_mesh():
    return plsc.VectorSubcoreMesh(core_axis_name="c", subcore_axis_name="s")


@functools.cache
def _get_deg_kernel():
    return functools.partial(
        pl.kernel,
        out_type=jax.ShapeDtypeStruct((NC, NV_PAD, 16), jnp.float32),
        mesh=_sc_mesh(),
        compiler_params=pltpu.CompilerParams(use_tc_tiling_on_sc=False),
        scratch_types=[
            pltpu.VMEM((2, 4, 128), jnp.int32),
            pltpu.VMEM((128, 16), jnp.float32),
            pltpu.VMEM((3136, 16), jnp.float32),
            pltpu.VMEM_SHARED((NV_PAD, 16), jnp.float32),
            pltpu.SemaphoreType.DMA,
            pltpu.SemaphoreType.DMA,
        ],
    )(_deg_body)


def _deg_body(src_hbm, ones_hbm, z3_hbm, ds_out, sidx, ones, zbuf, accs,
              semi, sems):
    c = lax.axis_index("c")
    s = lax.axis_index("s")
    wid = s * NC + c
    pltpu.sync_copy(ones_hbm, ones)
    pltpu.sync_copy(z3_hbm, zbuf)
    pltpu.sync_copy(zbuf, accs.at[pl.ds(s * 3136, 3136)])
    plsc.subcore_barrier()
    base = wid * 196
    pltpu.async_copy(src_hbm.at[pl.ds(base, 4)], sidx.at[0], semi)

    def body(g, _):
        h = g % 2
        pltpu.make_async_copy(src_hbm.at[pl.ds(0, 4)], sidx.at[0], semi).wait()
        for j in range(4):
            pltpu.async_copy(ones, accs.at[sidx.at[h, j]], sems, add=True)

        @pl.when(g >= 1)
        def _():
            for j in range(4):
                pltpu.make_async_copy(ones, accs.at[sidx.at[0, j]],
                                      sems).wait()

        @pl.when(g < 48)
        def _():
            pltpu.async_copy(src_hbm.at[pl.ds(base + (g + 1) * 4, 4)],
                             sidx.at[1 - h], semi)

        return 0

    lax.fori_loop(0, 49, body, 0)
    for j in range(4):
        pltpu.make_async_copy(ones, accs.at[sidx.at[0, j]], sems).wait()
    plsc.subcore_barrier()
    pltpu.sync_copy(accs.at[pl.ds(s * 3136, 3136)],
                    ds_out.at[c, pl.ds(s * 3136, 3136)])


@functools.cache
def _get_enc_kernel():
    return functools.partial(
        pl.kernel,
        out_type=jax.ShapeDtypeStruct((NC, NHP_PAD, 16), jnp.float32),
        mesh=_sc_mesh(),
        compiler_params=pltpu.CompilerParams(use_tc_tiling_on_sc=False),
        scratch_types=[
            pltpu.VMEM((4, 4, 128), jnp.int32),
            pltpu.VMEM((4, 4, 128), jnp.int32),
            pltpu.VMEM((2, 4, 128, 16), jnp.float32),
            pltpu.VMEM((784, 16), jnp.float32),
            pltpu.VMEM_SHARED((NHP_PAD, 16), jnp.float32),
            pltpu.SemaphoreType.DMA,
            pltpu.SemaphoreType.DMA,
            pltpu.SemaphoreType.DMA,
        ],
    )(_enc_body)


def _enc_body(xs_hbm, src_hbm, dst_hbm, z2_hbm, enc_out, sidx, didx, rbuf,
              zbuf, acce, semi, semg, sems):
    c = lax.axis_index("c")
    s = lax.axis_index("s")
    wid = s * NC + c
    pltpu.sync_copy(z2_hbm, zbuf)
    pltpu.sync_copy(zbuf, acce.at[pl.ds(s * 784, 784)])
    plsc.subcore_barrier()
    base = wid * 196
    pltpu.async_copy(src_hbm.at[pl.ds(base, 4)], sidx.at[0], semi)
    pltpu.async_copy(dst_hbm.at[pl.ds(base, 4)], didx.at[0], semi)

    def body(g, _):
        h4 = g % 4
        hr = g % 2
        pltpu.make_async_copy(src_hbm.at[pl.ds(0, 4)], sidx.at[0], semi).wait()
        pltpu.make_async_copy(dst_hbm.at[pl.ds(0, 4)], didx.at[0], semi).wait()

        @pl.when(g >= 2)
        def _():
            for j in range(4):
                pltpu.make_async_copy(rbuf.at[0, j],
                                      acce.at[sidx.at[0, j]], sems).wait()

        for j in range(4):
            pltpu.async_copy(xs_hbm.at[sidx.at[h4, j]], rbuf.at[hr, j], semg)

        @pl.when(g < 48)
        def _():
            r1 = base + (g + 1) * 4
            nb = (g + 1) % 4
            pltpu.async_copy(src_hbm.at[pl.ds(r1, 4)], sidx.at[nb], semi)
            pltpu.async_copy(dst_hbm.at[pl.ds(r1, 4)], didx.at[nb], semi)

        for j in range(4):
            pltpu.make_async_copy(xs_hbm.at[sidx.at[0, j]], rbuf.at[0, j],
                                  semg).wait()
        for j in range(4):
            pltpu.async_copy(rbuf.at[hr, j], acce.at[didx.at[h4, j]], sems,
                             add=True)
        return 0

    lax.fori_loop(0, 49, body, 0)
    for j in range(8):
        pltpu.make_async_copy(rbuf.at[0, j % 4],
                              acce.at[sidx.at[0, j % 4]], sems).wait()
    plsc.subcore_barrier()
    pltpu.sync_copy(acce.at[pl.ds(s * 784, 784)],
                    enc_out.at[c, pl.ds(s * 784, 784)])


@functools.cache
def _get_dec_kernel():
    return functools.partial(
        pl.kernel,
        out_type=jax.ShapeDtypeStruct((16, NV_PAD, 16), jnp.float32),
        mesh=_sc_mesh(),
        compiler_params=pltpu.CompilerParams(use_tc_tiling_on_sc=False),
        scratch_types=[
            pltpu.VMEM((4, 4, 128), jnp.int32),
            pltpu.VMEM((4, 4, 128), jnp.int32),
            pltpu.VMEM((3, 4, 128, 16), jnp.float32),
            pltpu.VMEM((3136, 16), jnp.float32),
            pltpu.VMEM_SHARED((NV_PAD, 16), jnp.float32),
            pltpu.SemaphoreType.DMA,
            pltpu.SemaphoreType.DMA,
            pltpu.SemaphoreType.DMA,
        ],
    )(_dec_body)


def _dec_body(y_hbm, src_hbm, dst_hbm, z3_hbm, dec_out, sidx, didx, rbuf,
              zbuf, acc, semi, semg, sems):
    c = lax.axis_index("c")
    s = lax.axis_index("s")
    pltpu.sync_copy(z3_hbm, zbuf)
    for cc_local in range(8):
        cc = c * 8 + cc_local
        pltpu.sync_copy(zbuf, acc.at[pl.ds(s * 3136, 3136)])
        plsc.subcore_barrier()
        base = s * 392
        pltpu.async_copy(src_hbm.at[pl.ds(base, 4)], sidx.at[0], semi)
        pltpu.async_copy(dst_hbm.at[pl.ds(base, 4)], didx.at[0], semi)

        def body(g, _):
            h4 = g % 4
            h3 = g % 3
            pltpu.make_async_copy(src_hbm.at[pl.ds(0, 4)], sidx.at[0],
                                  semi).wait()
            pltpu.make_async_copy(dst_hbm.at[pl.ds(0, 4)], didx.at[0],
                                  semi).wait()

            @pl.when(g >= 1)
            def _():
                p4 = (g + 3) % 4
                p3 = (g + 2) % 3
                for j in range(4):
                    pltpu.make_async_copy(y_hbm.at[0].at[didx.at[0, j]],
                                          rbuf.at[0, j], semg).wait()
                for j in range(4):
                    pltpu.async_copy(rbuf.at[p3, j], acc.at[sidx.at[p4, j]],
                                     sems, add=True)

            @pl.when(g >= 3)
            def _():
                for j in range(4):
                    pltpu.make_async_copy(rbuf.at[0, j],
                                          acc.at[sidx.at[0, j]], sems).wait()

            for j in range(4):
                pltpu.async_copy(y_hbm.at[cc].at[didx.at[h4, j]],
                                 rbuf.at[h3, j], semg)

            @pl.when(g < 97)
            def _():
                r1 = base + (g + 1) * 4
                nb = (g + 1) % 4
                pltpu.async_copy(src_hbm.at[pl.ds(r1, 4)], sidx.at[nb],
                                 semi)
                pltpu.async_copy(dst_hbm.at[pl.ds(r1, 4)], didx.at[nb],
                                 semi)

            return 0

        lax.fori_loop(0, 98, body, 0)
        for j in range(4):
            pltpu.make_async_copy(y_hbm.at[0].at[didx.at[0, j]], rbuf.at[0, j],
                                  semg).wait()
        for j in range(4):
            pltpu.async_copy(rbuf.at[97 % 3, j], acc.at[sidx.at[97 % 4, j]],
                             sems, add=True)
        for j in range(12):
            pltpu.make_async_copy(rbuf.at[0, j % 4],
                                  acc.at[sidx.at[0, j % 4]], sems).wait()
        plsc.subcore_barrier()
        pltpu.sync_copy(acc.at[pl.ds(s * 3136, 3136)],
                        dec_out.at[cc, pl.ds(s * 3136, 3136)])
        plsc.subcore_barrier()


def _xs_body(degp_ref, xvt_ref, out_ref, iss_ref):
    deg = degp_ref[0, :, 0] + degp_ref[1, :, 0]
    is_src = lax.rsqrt(jnp.maximum(deg, 1.0))
    x = jnp.nan_to_num(xvt_ref[...])
    b = x.shape[0]
    xsc = x * is_src[:, None]
    out_ref[...] = jnp.concatenate(
        [xsc, jnp.ones((b, 1), jnp.float32),
         jnp.zeros((b, 11), jnp.float32)], axis=1)
    iss_ref[...] = is_src[:, None]


def _tc_xs(degp, xvt):
    b = 1568
    return pl.pallas_call(
        _xs_body,
        out_shape=(jax.ShapeDtypeStruct((NV_PAD, 16), jnp.float32),
                   jax.ShapeDtypeStruct((NV_PAD, 1), jnp.float32)),
        grid=(NV_PAD // b,),
        in_specs=[
            pl.BlockSpec((NC, b, 16), lambda i: (0, i, 0)),
            pl.BlockSpec((b, TT), lambda i: (i, 0)),
        ],
        out_specs=(pl.BlockSpec((b, 16), lambda i: (i, 0)),
                   pl.BlockSpec((b, 1), lambda i: (i, 0))),
    )(degp, xvt)


def _y_body(encp_ref, wenc_ref, benc_ref, wtime_ref, out_ref):
    ep = encp_ref[0] + encp_ref[1]
    agg = ep[:, :TT]
    degd = ep[:, TT]
    isd = lax.rsqrt(jnp.maximum(degd, 1.0))
    a = agg * isd[:, None]
    wenc = wenc_ref[...]
    benc = benc_ref[...]
    wt = wtime_ref[...]
    hs = []
    for t in range(TT):
        h = a[:, t][:, None] * wenc[t][None, :] + benc[t][None, :]
        hs.append(jnp.where(h > 0, h, 0.01 * h))
    for o in range(TT):
        y = hs[0] * wt[0, o]
        for t in range(1, TT):
            y = y + hs[t] * wt[t, o]
        y = y * isd[:, None]
        for k in range(4):
            out_ref[4 * o + k] = y[:, 16 * k:16 * (k + 1)]


def _tc_y(encp, wenc, benc, wtime):
    b = 1792
    return pl.pallas_call(
        _y_body,
        out_shape=jax.ShapeDtypeStruct((16, NHP_PAD, 16), jnp.float32),
        grid=(NHP_PAD // b,),
        in_specs=[
            pl.BlockSpec((NC, b, 16), lambda i: (0, i, 0)),
            pl.BlockSpec((TT, HD), lambda i: (0, 0)),
            pl.BlockSpec((TT, HD), lambda i: (0, 0)),
            pl.BlockSpec((TT, TT), lambda i: (0, 0)),
        ],
        out_specs=pl.BlockSpec((16, b, 16), lambda i: (0, i, 0)),
    )(encp, wenc, benc, wtime)


def _out_body(dec_ref, iss_ref, xvt_ref, wdec_ref, bdec_ref, wlin_ref,
              blin_ref, watt1_ref, batt1_ref, watt2_ref, batt2_ref, wout_ref,
              bout_ref, out_ref):
    iss = iss_ref[:, 0]
    x = jnp.nan_to_num(xvt_ref[...])
    watt1 = watt1_ref[...]
    batt1 = batt1_ref[...]
    watt2 = watt2_ref[...]
    batt2 = batt2_ref[0]
    wlin = wlin_ref[...]
    blin = blin_ref[...]
    wout = wout_ref[...]
    bout = bout_ref[0]
    outs = []
    for t in range(TT):
        agg = jnp.concatenate([dec_ref[4 * t + k] for k in range(4)],
                              axis=1)
        agg = agg * iss[:, None]
        s1 = jnp.dot(agg, wdec_ref[t], preferred_element_type=jnp.float32)
        s1 = s1 + bdec_ref[t][None, :]
        s1 = jnp.where(s1 > 0, s1, 0.01 * s1)
        s2 = x[:, t][:, None] * wlin[None, :] + blin[None, :]
        t1 = jnp.tanh(jnp.dot(s1, watt1, preferred_element_type=jnp.float32)
                      + batt1[None, :])
        t2 = jnp.tanh(jnp.dot(s2, watt1, preferred_element_type=jnp.float32)
                      + batt1[None, :])
        a1 = jnp.sum(t1 * watt2[None, :], axis=1) + batt2
        a2 = jnp.sum(t2 * watt2[None, :], axis=1) + batt2
        m = jnp.maximum(a1, a2)
        e1 = jnp.exp(a1 - m)
        e2 = jnp.exp(a2 - m)
        inv = 1.0 / (e1 + e2)
        al1 = (e1 * inv)[:, None]
        al2 = (e2 * inv)[:, None]
        pooled = al1 * s1 + al2 * s2
        outs.append(jnp.sum(pooled * wout[None, :], axis=1) + bout)
    out_ref[...] = jnp.stack(outs, axis=0)[:, :, None]


def _tc_out(dec, iss, xvt, wdec, bdec, wlin, blin, watt1, batt1, watt2,
            batt2, wout, bout):
    b = 1568
    full = lambda shape: pl.BlockSpec(shape, lambda i: tuple(0 for _ in shape))
    return pl.pallas_call(
        _out_body,
        out_shape=jax.ShapeDtypeStruct((TT, NV_PAD, 1), jnp.float32),
        grid=(NV_PAD // b,),
        in_specs=[
            pl.BlockSpec((16, b, 16), lambda i: (0, i, 0)),
            pl.BlockSpec((b, 1), lambda i: (i, 0)),
            pl.BlockSpec((b, TT), lambda i: (i, 0)),
            full((TT, HD, HD)),
            full((TT, HD)),
            full((HD,)),
            full((HD,)),
            full((HD, 2 * HD)),
            full((2 * HD,)),
            full((2 * HD,)),
            full((1,)),
            full((HD,)),
            full((1,)),
        ],
        out_specs=pl.BlockSpec((TT, b, 1), lambda i: (0, i, 0)),
    )(dec, iss, xvt, wdec, bdec, wlin, blin, watt1, batt1, watt2, batt2,
      wout, bout)


def kernel(x_var, edge_src, edge_dst, W_enc, b_enc, W_time, W_dec, b_dec,
           W_lin, b_lin, W_att1, b_att1, W_att2, b_att2, W_out, b_out):
    pad_n = E_PAD - NE
    pad_ids = jnp.arange(pad_n, dtype=jnp.int32)
    src2d = jnp.concatenate(
        [edge_src, NV + (pad_ids % 64)]).reshape(ER, 128)
    dst2d = jnp.concatenate(
        [edge_dst, NHP + (pad_ids % 64)]).reshape(ER, 128)
    xvt = jnp.zeros((NV_PAD, TT), jnp.float32)
    xvt = xvt.at[:NV].set(x_var[:, :, 0].T)
    z2 = jnp.zeros((784, 16), jnp.float32)
    z3 = jnp.zeros((3136, 16), jnp.float32)
    ones128 = jnp.ones((128, 16), jnp.float32)

    ds_p = _get_deg_kernel()(src2d, ones128, z3)
    xs, iss = _tc_xs(ds_p, xvt)
    enc_p = _get_enc_kernel()(xs, src2d, dst2d, z2)
    y = _tc_y(enc_p, W_enc[:, 0, :], b_enc, W_time)
    dec = _get_dec_kernel()(y, src2d, dst2d, z3)
    out = _tc_out(dec, iss, xvt, W_dec, b_dec, W_lin[0], b_lin, W_att1,
                  b_att1, W_att2[:, 0], b_att2, W_out[:, 0], b_out)
    return out[:, :NV]


# speedup vs baseline: 24.0229x; 1.0032x over previous
import functools

import jax
import jax.numpy as jnp
from jax import lax
from jax.experimental import pallas as pl
from jax.experimental.pallas import tpu as pltpu
from jax.experimental.pallas import tpu_sc as plsc

NV = 50000
NHP = 12288
NE = 800000
HD = 64
TT = 4

NC = 2
NS = 16

NV_PAD = 50176
NHP_PAD = 12544
E_PAD = 802816
ER = E_PAD // 128


def _sc_mesh():
    return plsc.VectorSubcoreMesh(core_axis_name="c", subcore_axis_name="s")


@functools.cache
def _get_deg_kernel():
    return functools.partial(
        pl.kernel,
        out_type=jax.ShapeDtypeStruct((NC, NV_PAD, 16), jnp.float32),
        mesh=_sc_mesh(),
        compiler_params=pltpu.CompilerParams(use_tc_tiling_on_sc=False),
        scratch_types=[
            pltpu.VMEM((2, 4, 128), jnp.int32),
            pltpu.VMEM((128, 16), jnp.float32),
            pltpu.VMEM((3136, 16), jnp.float32),
            pltpu.VMEM_SHARED((NV_PAD, 16), jnp.float32),
            pltpu.SemaphoreType.DMA,
            pltpu.SemaphoreType.DMA,
        ],
    )(_deg_body)


def _deg_body(src_hbm, ones_hbm, z3_hbm, ds_out, sidx, ones, zbuf, accs,
              semi, sems):
    c = lax.axis_index("c")
    s = lax.axis_index("s")
    wid = s * NC + c
    pltpu.sync_copy(ones_hbm, ones)
    pltpu.sync_copy(z3_hbm, zbuf)
    pltpu.sync_copy(zbuf, accs.at[pl.ds(s * 3136, 3136)])
    plsc.subcore_barrier()
    base = wid * 196
    pltpu.async_copy(src_hbm.at[pl.ds(base, 4)], sidx.at[0], semi)

    def body(g, _):
        h = g % 2
        pltpu.make_async_copy(src_hbm.at[pl.ds(0, 4)], sidx.at[0], semi).wait()
        for j in range(4):
            pltpu.async_copy(ones, accs.at[sidx.at[h, j]], sems, add=True)

        @pl.when(g >= 1)
        def _():
            for j in range(4):
                pltpu.make_async_copy(ones, accs.at[sidx.at[0, j]],
                                      sems).wait()

        @pl.when(g < 48)
        def _():
            pltpu.async_copy(src_hbm.at[pl.ds(base + (g + 1) * 4, 4)],
                             sidx.at[1 - h], semi)

        return 0

    lax.fori_loop(0, 49, body, 0)
    for j in range(4):
        pltpu.make_async_copy(ones, accs.at[sidx.at[0, j]], sems).wait()
    plsc.subcore_barrier()
    pltpu.sync_copy(accs.at[pl.ds(s * 3136, 3136)],
                    ds_out.at[c, pl.ds(s * 3136, 3136)])


@functools.cache
def _get_enc_kernel():
    return functools.partial(
        pl.kernel,
        out_type=jax.ShapeDtypeStruct((NC, NHP_PAD, 16), jnp.float32),
        mesh=_sc_mesh(),
        compiler_params=pltpu.CompilerParams(use_tc_tiling_on_sc=False),
        scratch_types=[
            pltpu.VMEM((4, 4, 128), jnp.int32),
            pltpu.VMEM((4, 4, 128), jnp.int32),
            pltpu.VMEM((2, 4, 128, 16), jnp.float32),
            pltpu.VMEM((784, 16), jnp.float32),
            pltpu.VMEM_SHARED((NHP_PAD, 16), jnp.float32),
            pltpu.SemaphoreType.DMA,
            pltpu.SemaphoreType.DMA,
            pltpu.SemaphoreType.DMA,
        ],
    )(_enc_body)


def _enc_body(xs_hbm, src_hbm, dst_hbm, z2_hbm, enc_out, sidx, didx, rbuf,
              zbuf, acce, semi, semg, sems):
    c = lax.axis_index("c")
    s = lax.axis_index("s")
    wid = s * NC + c
    pltpu.sync_copy(z2_hbm, zbuf)
    pltpu.sync_copy(zbuf, acce.at[pl.ds(s * 784, 784)])
    plsc.subcore_barrier()
    base = wid * 196
    pltpu.async_copy(src_hbm.at[pl.ds(base, 4)], sidx.at[0], semi)
    pltpu.async_copy(dst_hbm.at[pl.ds(base, 4)], didx.at[0], semi)

    def body(g, _):
        h4 = g % 4
        hr = g % 2
        pltpu.make_async_copy(src_hbm.at[pl.ds(0, 4)], sidx.at[0], semi).wait()
        pltpu.make_async_copy(dst_hbm.at[pl.ds(0, 4)], didx.at[0], semi).wait()

        @pl.when(g >= 2)
        def _():
            for j in range(4):
                pltpu.make_async_copy(rbuf.at[0, j],
                                      acce.at[sidx.at[0, j]], sems).wait()

        for j in range(4):
            pltpu.async_copy(xs_hbm.at[sidx.at[h4, j]], rbuf.at[hr, j], semg)

        @pl.when(g < 48)
        def _():
            r1 = base + (g + 1) * 4
            nb = (g + 1) % 4
            pltpu.async_copy(src_hbm.at[pl.ds(r1, 4)], sidx.at[nb], semi)
            pltpu.async_copy(dst_hbm.at[pl.ds(r1, 4)], didx.at[nb], semi)

        for j in range(4):
            pltpu.make_async_copy(xs_hbm.at[sidx.at[0, j]], rbuf.at[0, j],
                                  semg).wait()
        for j in range(4):
            pltpu.async_copy(rbuf.at[hr, j], acce.at[didx.at[h4, j]], sems,
                             add=True)
        return 0

    lax.fori_loop(0, 49, body, 0)
    for j in range(8):
        pltpu.make_async_copy(rbuf.at[0, j % 4],
                              acce.at[sidx.at[0, j % 4]], sems).wait()
    plsc.subcore_barrier()
    pltpu.sync_copy(acce.at[pl.ds(s * 784, 784)],
                    enc_out.at[c, pl.ds(s * 784, 784)])


@functools.cache
def _get_dec_kernel():
    return functools.partial(
        pl.kernel,
        out_type=jax.ShapeDtypeStruct((16, NV_PAD, 16), jnp.float32),
        mesh=_sc_mesh(),
        compiler_params=pltpu.CompilerParams(use_tc_tiling_on_sc=False),
        scratch_types=[
            pltpu.VMEM((4, 1, 512), jnp.int32),
            pltpu.VMEM((4, 1, 512), jnp.int32),
            pltpu.VMEM((3, 512, 16), jnp.float32),
            pltpu.VMEM((3136, 16), jnp.float32),
            pltpu.VMEM_SHARED((NV_PAD, 16), jnp.float32),
            pltpu.SemaphoreType.DMA,
            pltpu.SemaphoreType.DMA,
            pltpu.SemaphoreType.DMA,
        ],
    )(_dec_body)


def _dec_body(y_hbm, src_hbm, dst_hbm, z3_hbm, dec_out, sidx, didx, rbuf,
              zbuf, acc, semi, semg, sems):
    c = lax.axis_index("c")
    s = lax.axis_index("s")
    pltpu.sync_copy(z3_hbm, zbuf)
    for cc_local in range(8):
        cc = c * 8 + cc_local
        pltpu.sync_copy(zbuf, acc.at[pl.ds(s * 3136, 3136)])
        plsc.subcore_barrier()
        base = s * 98
        pltpu.async_copy(src_hbm.at[pl.ds(base, 1)], sidx.at[0], semi)
        pltpu.async_copy(dst_hbm.at[pl.ds(base, 1)], didx.at[0], semi)

        def body(g, _):
            h4 = g % 4
            h3 = g % 3
            pltpu.make_async_copy(src_hbm.at[pl.ds(0, 1)], sidx.at[0],
                                  semi).wait()
            pltpu.make_async_copy(dst_hbm.at[pl.ds(0, 1)], didx.at[0],
                                  semi).wait()

            @pl.when(g >= 1)
            def _():
                p4 = (g + 3) % 4
                p3 = (g + 2) % 3
                pltpu.make_async_copy(y_hbm.at[0].at[didx.at[0, 0]],
                                      rbuf.at[0], semg).wait()
                pltpu.async_copy(rbuf.at[p3], acc.at[sidx.at[p4, 0]],
                                 sems, add=True)

            @pl.when(g >= 3)
            def _():
                pltpu.make_async_copy(rbuf.at[0],
                                      acc.at[sidx.at[0, 0]], sems).wait()

            pltpu.async_copy(y_hbm.at[cc].at[didx.at[h4, 0]], rbuf.at[h3],
                             semg)

            @pl.when(g < 97)
            def _():
                r1 = base + g + 1
                nb = (g + 1) % 4
                pltpu.async_copy(src_hbm.at[pl.ds(r1, 1)], sidx.at[nb],
                                 semi)
                pltpu.async_copy(dst_hbm.at[pl.ds(r1, 1)], didx.at[nb],
                                 semi)

            return 0

        lax.fori_loop(0, 98, body, 0)
        pltpu.make_async_copy(y_hbm.at[0].at[didx.at[0, 0]], rbuf.at[0],
                              semg).wait()
        pltpu.async_copy(rbuf.at[97 % 3], acc.at[sidx.at[97 % 4, 0]],
                         sems, add=True)
        for j in range(3):
            pltpu.make_async_copy(rbuf.at[0],
                                  acc.at[sidx.at[0, 0]], sems).wait()
        plsc.subcore_barrier()
        pltpu.sync_copy(acc.at[pl.ds(s * 3136, 3136)],
                        dec_out.at[cc, pl.ds(s * 3136, 3136)])
        plsc.subcore_barrier()


def _xs_body(degp_ref, xvt_ref, out_ref, iss_ref):
    deg = degp_ref[0, :, 0] + degp_ref[1, :, 0]
    is_src = lax.rsqrt(jnp.maximum(deg, 1.0))
    x = jnp.nan_to_num(xvt_ref[...])
    b = x.shape[0]
    xsc = x * is_src[:, None]
    out_ref[...] = jnp.concatenate(
        [xsc, jnp.ones((b, 1), jnp.float32),
         jnp.zeros((b, 11), jnp.float32)], axis=1)
    iss_ref[...] = is_src[:, None]


def _tc_xs(degp, xvt):
    b = 1568
    return pl.pallas_call(
        _xs_body,
        out_shape=(jax.ShapeDtypeStruct((NV_PAD, 16), jnp.float32),
                   jax.ShapeDtypeStruct((NV_PAD, 1), jnp.float32)),
        grid=(NV_PAD // b,),
        in_specs=[
            pl.BlockSpec((NC, b, 16), lambda i: (0, i, 0)),
            pl.BlockSpec((b, TT), lambda i: (i, 0)),
        ],
        out_specs=(pl.BlockSpec((b, 16), lambda i: (i, 0)),
                   pl.BlockSpec((b, 1), lambda i: (i, 0))),
    )(degp, xvt)


def _y_body(encp_ref, wenc_ref, benc_ref, wtime_ref, out_ref):
    ep = encp_ref[0] + encp_ref[1]
    agg = ep[:, :TT]
    degd = ep[:, TT]
    isd = lax.rsqrt(jnp.maximum(degd, 1.0))
    a = agg * isd[:, None]
    wenc = wenc_ref[...]
    benc = benc_ref[...]
    wt = wtime_ref[...]
    hs = []
    for t in range(TT):
        h = a[:, t][:, None] * wenc[t][None, :] + benc[t][None, :]
        hs.append(jnp.where(h > 0, h, 0.01 * h))
    for o in range(TT):
        y = hs[0] * wt[0, o]
        for t in range(1, TT):
            y = y + hs[t] * wt[t, o]
        y = y * isd[:, None]
        for k in range(4):
            out_ref[4 * o + k] = y[:, 16 * k:16 * (k + 1)]


def _tc_y(encp, wenc, benc, wtime):
    b = 1792
    return pl.pallas_call(
        _y_body,
        out_shape=jax.ShapeDtypeStruct((16, NHP_PAD, 16), jnp.float32),
        grid=(NHP_PAD // b,),
        in_specs=[
            pl.BlockSpec((NC, b, 16), lambda i: (0, i, 0)),
            pl.BlockSpec((TT, HD), lambda i: (0, 0)),
            pl.BlockSpec((TT, HD), lambda i: (0, 0)),
            pl.BlockSpec((TT, TT), lambda i: (0, 0)),
        ],
        out_specs=pl.BlockSpec((16, b, 16), lambda i: (0, i, 0)),
    )(encp, wenc, benc, wtime)


def _out_body(dec_ref, iss_ref, xvt_ref, wdec_ref, bdec_ref, wlin_ref,
              blin_ref, watt1_ref, batt1_ref, watt2_ref, batt2_ref, wout_ref,
              bout_ref, out_ref):
    iss = iss_ref[:, 0]
    x = jnp.nan_to_num(xvt_ref[...])
    watt1 = watt1_ref[...]
    batt1 = batt1_ref[...]
    watt2 = watt2_ref[...]
    batt2 = batt2_ref[0]
    wlin = wlin_ref[...]
    blin = blin_ref[...]
    wout = wout_ref[...]
    bout = bout_ref[0]
    outs = []
    for t in range(TT):
        agg = jnp.concatenate([dec_ref[4 * t + k] for k in range(4)],
                              axis=1)
        agg = agg * iss[:, None]
        s1 = jnp.dot(agg, wdec_ref[t], preferred_element_type=jnp.float32)
        s1 = s1 + bdec_ref[t][None, :]
        s1 = jnp.where(s1 > 0, s1, 0.01 * s1)
        s2 = x[:, t][:, None] * wlin[None, :] + blin[None, :]
        t1 = jnp.tanh(jnp.dot(s1, watt1, preferred_element_type=jnp.float32)
                      + batt1[None, :])
        t2 = jnp.tanh(jnp.dot(s2, watt1, preferred_element_type=jnp.float32)
                      + batt1[None, :])
        a1 = jnp.sum(t1 * watt2[None, :], axis=1) + batt2
        a2 = jnp.sum(t2 * watt2[None, :], axis=1) + batt2
        m = jnp.maximum(a1, a2)
        e1 = jnp.exp(a1 - m)
        e2 = jnp.exp(a2 - m)
        inv = 1.0 / (e1 + e2)
        al1 = (e1 * inv)[:, None]
        al2 = (e2 * inv)[:, None]
        pooled = al1 * s1 + al2 * s2
        outs.append(jnp.sum(pooled * wout[None, :], axis=1) + bout)
    out_ref[...] = jnp.stack(outs, axis=0)[:, :, None]


def _tc_out(dec, iss, xvt, wdec, bdec, wlin, blin, watt1, batt1, watt2,
            batt2, wout, bout):
    b = 1568
    full = lambda shape: pl.BlockSpec(shape, lambda i: tuple(0 for _ in shape))
    return pl.pallas_call(
        _out_body,
        out_shape=jax.ShapeDtypeStruct((TT, NV_PAD, 1), jnp.float32),
        grid=(NV_PAD // b,),
        in_specs=[
            pl.BlockSpec((16, b, 16), lambda i: (0, i, 0)),
            pl.BlockSpec((b, 1), lambda i: (i, 0)),
            pl.BlockSpec((b, TT), lambda i: (i, 0)),
            full((TT, HD, HD)),
            full((TT, HD)),
            full((HD,)),
            full((HD,)),
            full((HD, 2 * HD)),
            full((2 * HD,)),
            full((2 * HD,)),
            full((1,)),
            full((HD,)),
            full((1,)),
        ],
        out_specs=pl.BlockSpec((TT, b, 1), lambda i: (0, i, 0)),
    )(dec, iss, xvt, wdec, bdec, wlin, blin, watt1, batt1, watt2, batt2,
      wout, bout)


def kernel(x_var, edge_src, edge_dst, W_enc, b_enc, W_time, W_dec, b_dec,
           W_lin, b_lin, W_att1, b_att1, W_att2, b_att2, W_out, b_out):
    pad_n = E_PAD - NE
    pad_ids = jnp.arange(pad_n, dtype=jnp.int32)
    src2d = jnp.concatenate(
        [edge_src, NV + (pad_ids % 64)]).reshape(ER, 128)
    dst2d = jnp.concatenate(
        [edge_dst, NHP + (pad_ids % 64)]).reshape(ER, 128)
    xvt = jnp.zeros((NV_PAD, TT), jnp.float32)
    xvt = xvt.at[:NV].set(x_var[:, :, 0].T)
    z2 = jnp.zeros((784, 16), jnp.float32)
    z3 = jnp.zeros((3136, 16), jnp.float32)
    ones128 = jnp.ones((128, 16), jnp.float32)

    ds_p = _get_deg_kernel()(src2d, ones128, z3)
    xs, iss = _tc_xs(ds_p, xvt)
    enc_p = _get_enc_kernel()(xs, src2d, dst2d, z2)
    y = _tc_y(enc_p, W_enc[:, 0, :], b_enc, W_time)
    dec = _get_dec_kernel()(y, src2d.reshape(ER // 4, 512),
                            dst2d.reshape(ER // 4, 512), z3)
    out = _tc_out(dec, iss, xvt, W_dec, b_dec, W_lin[0], b_lin, W_att1,
                  b_att1, W_att2[:, 0], b_att2, W_out[:, 0], b_out)
    return out[:, :NV]
